# Initial kernel scaffold; baseline (speedup 1.0000x reference)
import jax
import jax.numpy as jnp
from jax.experimental import pallas as pl


def kernel(x, wg, w1, w2):
    raise NotImplementedError("write your pallas kernel here")


# speedup vs baseline: 1.9510x; 1.9510x over previous
import jax
import jax.numpy as jnp
from jax.experimental import pallas as pl

D_MODEL = 1024
D_FF = 4096
N_EXPERTS = 8
T = 2048
F_BLK = 512


def _gate_body(x_ref, wg_ref, comb_ref, route_ref):
    l = jnp.dot(x_ref[...], wg_ref[...], preferred_element_type=jnp.float32)
    lane = jax.lax.broadcasted_iota(jnp.int32, l.shape, 1)
    valid = lane < N_EXPERTS
    l = jnp.where(valid, l, -1e30)
    m = jnp.max(l, axis=1, keepdims=True)
    p = jnp.exp(l - m)
    p = jnp.where(valid, p, 0.0)
    probs = p / jnp.sum(p, axis=1, keepdims=True)
    m1 = jnp.max(probs, axis=1, keepdims=True)
    i1 = jnp.min(jnp.where(probs == m1, lane, 128), axis=1, keepdims=True)
    pr2 = jnp.where(lane == i1, -1.0, probs)
    m2 = jnp.max(pr2, axis=1, keepdims=True)
    i2 = jnp.min(jnp.where(pr2 == m2, lane, 128), axis=1, keepdims=True)
    den = m1 + m2 + 1e-9
    g1 = m1 / den
    g2 = m2 / den
    comb_ref[...] = jnp.where(lane == i1, g1, 0.0) + jnp.where(lane == i2, g2, 0.0)
    route_ref[...] = (
        jnp.where(lane == 0, i1.astype(jnp.float32), 0.0)
        + jnp.where(lane == 1, i2.astype(jnp.float32), 0.0)
        + jnp.where(lane == 2, g1, 0.0)
        + jnp.where(lane == 3, g2, 0.0)
    )


def _dense_body(comb_ref, x_ref, w1_ref, w2_ref, o_ref):
    e = pl.program_id(0)
    j = pl.program_id(1)

    @pl.when((e == 0) & (j == 0))
    def _():
        o_ref[...] = jnp.zeros_like(o_ref)

    h = jnp.dot(x_ref[...], w1_ref[0], preferred_element_type=jnp.float32)
    h = 0.5 * h * (1.0 + jax.lax.erf(h * 0.7071067811865476))
    lane = jax.lax.broadcasted_iota(jnp.int32, comb_ref.shape, 1)
    c = jnp.sum(jnp.where(lane == e, comb_ref[...], 0.0), axis=1, keepdims=True)
    y = jnp.dot(h.astype(jnp.bfloat16), w2_ref[0], preferred_element_type=jnp.float32)
    o_ref[...] += y * c


def kernel(x, wg, w1, w2):
    wg_p = jnp.zeros((D_MODEL, 128), jnp.float32).at[:, :N_EXPERTS].set(wg)
    comb, _route = pl.pallas_call(
        _gate_body,
        out_shape=[
            jax.ShapeDtypeStruct((T, 128), jnp.float32),
            jax.ShapeDtypeStruct((T, 128), jnp.float32),
        ],
    )(x, wg_p)

    xb = x.astype(jnp.bfloat16)
    w1b = w1.astype(jnp.bfloat16)
    w2b = w2.astype(jnp.bfloat16)
    nj = D_FF // F_BLK
    out = pl.pallas_call(
        _dense_body,
        grid=(N_EXPERTS, nj),
        in_specs=[
            pl.BlockSpec((T, 128), lambda e, j: (0, 0)),
            pl.BlockSpec((T, D_MODEL), lambda e, j: (0, 0)),
            pl.BlockSpec((1, D_MODEL, F_BLK), lambda e, j: (e, 0, j)),
            pl.BlockSpec((1, F_BLK, D_MODEL), lambda e, j: (e, j, 0)),
        ],
        out_specs=pl.BlockSpec((T, D_MODEL), lambda e, j: (0, 0)),
        out_shape=jax.ShapeDtypeStruct((T, D_MODEL), jnp.float32),
    )(comb, xb, w1b, w2b)
    return out
